# Initial kernel scaffold; baseline (speedup 1.0000x reference)
import jax
import jax.numpy as jnp
from jax.experimental import pallas as pl


def kernel(xs, src, dst, guards, initial_weights, final_weights):
    raise NotImplementedError("write your pallas kernel here")


# speedup vs baseline: 8.4351x; 8.4351x over previous
import functools

import jax
import jax.numpy as jnp
from jax import lax
from jax.experimental import pallas as pl
from jax.experimental.pallas import tpu as pltpu
from jax.experimental.pallas import tpu_sc as plsc

_NUM_WORKERS = 32
_LANES = 16
_BLK = 2048


def _coef_sparsecore(src_pad, dst_pad, initial_weights, final_weights):
    e_pad = src_pad.shape[0]
    q = initial_weights.shape[0]
    chunk = e_pad // _NUM_WORKERS
    mesh = plsc.VectorSubcoreMesh(core_axis_name="c", subcore_axis_name="s")

    @functools.partial(
        pl.kernel,
        out_type=jax.ShapeDtypeStruct((e_pad,), jnp.float32),
        mesh=mesh,
        compiler_params=pltpu.CompilerParams(needs_layout_passes=False),
        scratch_types=[
            pltpu.VMEM((chunk,), jnp.int32),
            pltpu.VMEM((chunk,), jnp.int32),
            pltpu.VMEM((chunk,), jnp.float32),
            pltpu.VMEM((q,), jnp.float32),
            pltpu.VMEM((q,), jnp.float32),
        ],
    )
    def sc_kernel(src_hbm, dst_hbm, ini_hbm, fin_hbm, coef_hbm,
                  src_v, dst_v, coef_v, ini_v, fin_v):
        wid = lax.axis_index("s") * 2 + lax.axis_index("c")
        base = wid * chunk
        pltpu.sync_copy(ini_hbm, ini_v)
        pltpu.sync_copy(fin_hbm, fin_v)
        pltpu.sync_copy(src_hbm.at[pl.ds(base, chunk)], src_v)
        pltpu.sync_copy(dst_hbm.at[pl.ds(base, chunk)], dst_v)

        def body(j, carry):
            o = j * _LANES
            s_idx = src_v[pl.ds(o, _LANES)]
            d_idx = dst_v[pl.ds(o, _LANES)]
            a = plsc.load_gather(ini_v, [s_idx])
            b = plsc.load_gather(fin_v, [d_idx])
            coef_v[pl.ds(o, _LANES)] = a * b
            return carry

        lax.fori_loop(0, chunk // _LANES, body, 0)
        pltpu.sync_copy(coef_v, coef_hbm.at[pl.ds(base, chunk)])

    return sc_kernel(src_pad, dst_pad, initial_weights, final_weights)


def _reduce_tensorcore(guards, xs_pad, coef3d, e_total):
    e_rows, d = guards.shape
    xp = xs_pad.shape[0]
    grid = coef3d.shape[0]

    def body(guards_ref, xs_ref, coef_ref, out_ref):
        i = pl.program_id(0)

        @pl.when(i == 0)
        def _init():
            out_ref[...] = jnp.zeros_like(out_ref)

        g = guards_ref[...]
        x = xs_ref[...]
        logits = lax.dot_general(
            x, g, (((1,), (1,)), ((), ())),
            preferred_element_type=jnp.float32)
        s = jax.nn.sigmoid(logits)
        c = coef_ref[...].reshape(1, _BLK)
        e_ids = i * _BLK + lax.broadcasted_iota(jnp.int32, (1, _BLK), 1)
        w = jnp.where(e_ids < e_total, s * c, 0.0)
        partial = jnp.sum(w, axis=1, keepdims=True)
        out_ref[...] += jnp.broadcast_to(partial, out_ref.shape)

    return pl.pallas_call(
        body,
        grid=(grid,),
        in_specs=[
            pl.BlockSpec((_BLK, d), lambda i: (i, 0)),
            pl.BlockSpec((xp, d), lambda i: (0, 0)),
            pl.BlockSpec((1, 1, _BLK), lambda i: (i, 0, 0)),
        ],
        out_specs=pl.BlockSpec((xp, 128), lambda i: (0, 0)),
        out_shape=jax.ShapeDtypeStruct((xp, 128), jnp.float32),
    )(guards, xs_pad, coef3d)


def kernel(xs, src, dst, guards, initial_weights, final_weights):
    b, d = xs.shape
    e = src.shape[0]
    grid = -(-e // _BLK)
    e_pad = grid * _BLK
    assert e_pad % (_NUM_WORKERS * _LANES) == 0

    src_pad = jnp.pad(src, (0, e_pad - e))
    dst_pad = jnp.pad(dst, (0, e_pad - e))
    coef = _coef_sparsecore(src_pad, dst_pad, initial_weights, final_weights)
    coef3d = coef.reshape(grid, 1, _BLK)

    xp = max(8, b)
    xs_pad = jnp.pad(xs, ((0, xp - b), (0, 0)))
    out2d = _reduce_tensorcore(guards, xs_pad, coef3d, e)
    return out2d[:b, 0]
